# Initial kernel scaffold; baseline (speedup 1.0000x reference)
import jax
import jax.numpy as jnp
from jax.experimental import pallas as pl


def kernel(node_embedding_matrix, batch_x_index, edge_index):
    raise NotImplementedError("write your pallas kernel here")


# speedup vs baseline: 5.2745x; 5.2745x over previous
import jax
import jax.numpy as jnp
from jax import lax
from jax.experimental import pallas as pl
from jax.experimental.pallas import tpu as pltpu
from jax.experimental.pallas import tpu_sc as plsc

N_NODES = 10000
D = 128
N_EDGES = 320000
C = 128
N_CHUNKS = N_EDGES // C
N_WORKERS = 32
BASE_CHUNKS = N_CHUNKS // N_WORKERS
EXTRA = N_CHUNKS - BASE_CHUNKS * N_WORKERS
GROUPS = C // 16


def _sc_kernel(table_hbm, edges_hbm, out_hbm, idx_v, s_rows, d_rows, out_v,
               sem_s, sem_d):
    nc = 2
    wid = lax.axis_index("s") * nc + lax.axis_index("c")
    my_chunks = BASE_CHUNKS + jnp.where(wid < EXTRA, 1, 0)
    start = BASE_CHUNKS * wid + jnp.minimum(wid, EXTRA)

    lane = lax.iota(jnp.int32, 16)

    def chunk_body(c, carry):
        base = (start + c) * C
        pltpu.sync_copy(edges_hbm.at[:, pl.ds(base, C)], idx_v)
        cp_s = pltpu.async_copy(table_hbm.at[idx_v.at[0]], s_rows, sem_s)
        cp_d = pltpu.async_copy(table_hbm.at[idx_v.at[1]], d_rows, sem_d)
        cp_s.wait()
        cp_d.wait()
        for g in range(GROUPS):
            row_idx = g * 16 + lane

            def jstep(j, acc):
                col = (j & ~15) + ((j + lane) & 15)
                s = plsc.load_gather(s_rows, [row_idx, col])
                t = plsc.load_gather(d_rows, [row_idx, col])
                return acc + s * t

            acc = lax.fori_loop(0, D, jstep, jnp.zeros((16,), jnp.float32),
                                unroll=8)
            p = 1.0 / (1.0 + jnp.exp(-acc))
            out_v[pl.ds(g * 16, 16)] = p
        pltpu.sync_copy(out_v, out_hbm.at[pl.ds(base, C)])
        return carry

    lax.fori_loop(0, my_chunks, chunk_body, 0)


@jax.jit
def _run(table, edge_index):
    mesh = plsc.VectorSubcoreMesh(core_axis_name="c", subcore_axis_name="s")
    kfn = pl.kernel(
        _sc_kernel,
        mesh=mesh,
        compiler_params=pltpu.CompilerParams(
            use_tc_tiling_on_sc=False, needs_layout_passes=False),
        out_type=jax.ShapeDtypeStruct((N_EDGES,), jnp.float32),
        scratch_types=[
            pltpu.VMEM((2, C), jnp.int32),
            pltpu.VMEM((C, D), jnp.float32),
            pltpu.VMEM((C, D), jnp.float32),
            pltpu.VMEM((C,), jnp.float32),
            pltpu.SemaphoreType.DMA,
            pltpu.SemaphoreType.DMA,
        ],
    )
    return kfn(table, edge_index)


def kernel(node_embedding_matrix, batch_x_index, edge_index):
    del batch_x_index
    return _run(node_embedding_matrix, edge_index)


# speedup vs baseline: 8.8917x; 1.6858x over previous
import jax
import jax.numpy as jnp
from jax import lax
from jax.experimental import pallas as pl
from jax.experimental.pallas import tpu as pltpu
from jax.experimental.pallas import tpu_sc as plsc

N_NODES = 10000
D = 128
N_EDGES = 320000
C = 128
N_CHUNKS = N_EDGES // C
N_WORKERS = 32
BASE_CHUNKS = N_CHUNKS // N_WORKERS
EXTRA = N_CHUNKS - BASE_CHUNKS * N_WORKERS
GROUPS = C // 16
PAIRS = BASE_CHUNKS // 2


def _sc_kernel(table_hbm, edges_hbm, out_hbm,
               idx0, idx1, s0, s1, d0, d1, o0, o1,
               sem_g, sem_idx, sem_out0, sem_out1):
    nc = 2
    wid = lax.axis_index("s") * nc + lax.axis_index("c")
    my_chunks = BASE_CHUNKS + jnp.where(wid < EXTRA, 1, 0)
    start = BASE_CHUNKS * wid + jnp.minimum(wid, EXTRA)

    lane = lax.iota(jnp.int32, 16)
    idx_b = (idx0, idx1)
    s_b = (s0, s1)
    d_b = (d0, d1)
    o_b = (o0, o1)

    def issue_idx(c, b):
        pltpu.async_copy(edges_hbm.at[:, pl.ds((start + c) * C, C)],
                         idx_b[b], sem_idx)

    def wait_idx(b):
        pltpu.make_async_copy(edges_hbm.at[:, pl.ds(0, C)], idx_b[b],
                              sem_idx).wait()

    def launch_gathers(b):
        pltpu.async_copy(table_hbm.at[idx_b[b].at[0]], s_b[b], sem_g)
        pltpu.async_copy(table_hbm.at[idx_b[b].at[1]], d_b[b], sem_g)

    def wait_gathers(b):
        pltpu.make_async_copy(table_hbm.at[pl.ds(0, C)], s_b[b], sem_g).wait()
        pltpu.make_async_copy(table_hbm.at[pl.ds(0, C)], d_b[b], sem_g).wait()

    sem_out = (sem_out0, sem_out1)

    def drain_out(b):
        pltpu.make_async_copy(o_b[b], out_hbm.at[pl.ds(0, C)],
                              sem_out[b]).wait()

    def compute(c, b):
        s_rows, d_rows, out_v = s_b[b], d_b[b], o_b[b]

        @pl.when(c >= 2)
        def _():
            drain_out(b)

        for g in range(GROUPS):
            row_idx = g * 16 + lane

            def jstep(j, acc):
                col = (j & ~15) + ((j + lane) & 15)
                s = plsc.load_gather(s_rows, [row_idx, col])
                t = plsc.load_gather(d_rows, [row_idx, col])
                return acc + s * t

            acc = lax.fori_loop(0, D, jstep, jnp.zeros((16,), jnp.float32),
                                unroll=8)
            p = 1.0 / (1.0 + jnp.exp(-acc))
            out_v[pl.ds(g * 16, 16)] = p

        pltpu.async_copy(out_v, out_hbm.at[pl.ds((start + c) * C, C)],
                         sem_out[b])

    def step(c, b, nb):
        wait_gathers(b)

        @pl.when(c + 1 < my_chunks)
        def _():
            wait_idx(nb)
            launch_gathers(nb)

        @pl.when(c + 2 < my_chunks)
        def _():
            issue_idx(c + 2, b)

        compute(c, b)

    pltpu.sync_copy(edges_hbm.at[:, pl.ds(start * C, C)], idx0)
    launch_gathers(0)
    issue_idx(1, 1)

    def pair_body(p, carry):
        step(2 * p, 0, 1)
        step(2 * p + 1, 1, 0)
        return carry

    lax.fori_loop(0, PAIRS, pair_body, 0)

    @pl.when(my_chunks > BASE_CHUNKS)
    def _():
        step(BASE_CHUNKS, 0, 1)

    drain_out(0)
    drain_out(1)


@jax.jit
def _run(table, edge_index):
    mesh = plsc.VectorSubcoreMesh(core_axis_name="c", subcore_axis_name="s")
    kfn = pl.kernel(
        _sc_kernel,
        mesh=mesh,
        compiler_params=pltpu.CompilerParams(
            use_tc_tiling_on_sc=False, needs_layout_passes=False),
        out_type=jax.ShapeDtypeStruct((N_EDGES,), jnp.float32),
        scratch_types=[
            pltpu.VMEM((2, C), jnp.int32),
            pltpu.VMEM((2, C), jnp.int32),
            pltpu.VMEM((C, D), jnp.float32),
            pltpu.VMEM((C, D), jnp.float32),
            pltpu.VMEM((C, D), jnp.float32),
            pltpu.VMEM((C, D), jnp.float32),
            pltpu.VMEM((C,), jnp.float32),
            pltpu.VMEM((C,), jnp.float32),
            pltpu.SemaphoreType.DMA,
            pltpu.SemaphoreType.DMA,
            pltpu.SemaphoreType.DMA,
            pltpu.SemaphoreType.DMA,
        ],
    )
    return kfn(table, edge_index)


def kernel(node_embedding_matrix, batch_x_index, edge_index):
    del batch_x_index
    return _run(node_embedding_matrix, edge_index)


# speedup vs baseline: 9.0878x; 1.0221x over previous
import jax
import jax.numpy as jnp
from jax import lax
from jax.experimental import pallas as pl
from jax.experimental.pallas import tpu as pltpu
from jax.experimental.pallas import tpu_sc as plsc

N_NODES = 10000
D = 128
N_EDGES = 320000
C = 64
N_CHUNKS = N_EDGES // C
N_WORKERS = 32
BASE_CHUNKS = N_CHUNKS // N_WORKERS
EXTRA = N_CHUNKS - BASE_CHUNKS * N_WORKERS
GROUPS = C // 16
PAIRS = BASE_CHUNKS // 2


def _sc_kernel(table_hbm, edges_hbm, out_hbm,
               table_sh, idx0, idx1, s0, s1, d0, d1, o0, o1,
               sem_g, sem_idx, sem_out0, sem_out1, sem_st):
    nc = 2
    wid = lax.axis_index("s") * nc + lax.axis_index("c")
    my_chunks = BASE_CHUNKS + jnp.where(wid < EXTRA, 1, 0)
    start = BASE_CHUNKS * wid + jnp.minimum(wid, EXTRA)

    sid = lax.axis_index("s")
    rows_per_sub = N_NODES // 16
    pltpu.async_copy(table_hbm.at[pl.ds(sid * rows_per_sub, rows_per_sub)],
                     table_sh.at[pl.ds(sid * rows_per_sub, rows_per_sub)],
                     sem_st).wait()
    plsc.subcore_barrier()

    lane = lax.iota(jnp.int32, 16)
    idx_b = (idx0, idx1)
    s_b = (s0, s1)
    d_b = (d0, d1)
    o_b = (o0, o1)

    def issue_idx(c, b):
        pltpu.async_copy(edges_hbm.at[:, pl.ds((start + c) * C, C)],
                         idx_b[b], sem_idx)

    def wait_idx(b):
        pltpu.make_async_copy(edges_hbm.at[:, pl.ds(0, C)], idx_b[b],
                              sem_idx).wait()

    def launch_gathers(b):
        pltpu.async_copy(table_sh.at[idx_b[b].at[0]], s_b[b], sem_g)
        pltpu.async_copy(table_sh.at[idx_b[b].at[1]], d_b[b], sem_g)

    def wait_gathers(b):
        pltpu.make_async_copy(table_hbm.at[pl.ds(0, C)], s_b[b], sem_g).wait()
        pltpu.make_async_copy(table_hbm.at[pl.ds(0, C)], d_b[b], sem_g).wait()

    sem_out = (sem_out0, sem_out1)

    def drain_out(b):
        pltpu.make_async_copy(o_b[b], out_hbm.at[pl.ds(0, C)],
                              sem_out[b]).wait()

    def compute(c, b):
        s_rows, d_rows, out_v = s_b[b], d_b[b], o_b[b]

        @pl.when(c >= 2)
        def _():
            drain_out(b)

        for g in range(GROUPS):
            row_idx = g * 16 + lane

            def jstep(j, acc):
                col = (j & ~15) + ((j + lane) & 15)
                s = plsc.load_gather(s_rows, [row_idx, col])
                t = plsc.load_gather(d_rows, [row_idx, col])
                return acc + s * t

            acc = lax.fori_loop(0, D, jstep, jnp.zeros((16,), jnp.float32),
                                unroll=8)
            p = 1.0 / (1.0 + jnp.exp(-acc))
            out_v[pl.ds(g * 16, 16)] = p

        pltpu.async_copy(out_v, out_hbm.at[pl.ds((start + c) * C, C)],
                         sem_out[b])

    def step(c, b, nb):
        wait_gathers(b)

        @pl.when(c + 1 < my_chunks)
        def _():
            wait_idx(nb)
            launch_gathers(nb)

        @pl.when(c + 2 < my_chunks)
        def _():
            issue_idx(c + 2, b)

        compute(c, b)

    pltpu.sync_copy(edges_hbm.at[:, pl.ds(start * C, C)], idx0)
    launch_gathers(0)
    issue_idx(1, 1)

    def pair_body(p, carry):
        step(2 * p, 0, 1)
        step(2 * p + 1, 1, 0)
        return carry

    lax.fori_loop(0, PAIRS, pair_body, 0)

    @pl.when(my_chunks > BASE_CHUNKS)
    def _():
        step(BASE_CHUNKS, 0, 1)

    drain_out(0)
    drain_out(1)


@jax.jit
def _run(table, edge_index):
    mesh = plsc.VectorSubcoreMesh(core_axis_name="c", subcore_axis_name="s")
    kfn = pl.kernel(
        _sc_kernel,
        mesh=mesh,
        compiler_params=pltpu.CompilerParams(
            use_tc_tiling_on_sc=False, needs_layout_passes=False),
        out_type=jax.ShapeDtypeStruct((N_EDGES,), jnp.float32),
        scratch_types=[
            pltpu.VMEM_SHARED((N_NODES, D), jnp.float32),
            pltpu.VMEM((2, C), jnp.int32),
            pltpu.VMEM((2, C), jnp.int32),
            pltpu.VMEM((C, D), jnp.float32),
            pltpu.VMEM((C, D), jnp.float32),
            pltpu.VMEM((C, D), jnp.float32),
            pltpu.VMEM((C, D), jnp.float32),
            pltpu.VMEM((C,), jnp.float32),
            pltpu.VMEM((C,), jnp.float32),
            pltpu.SemaphoreType.DMA,
            pltpu.SemaphoreType.DMA,
            pltpu.SemaphoreType.DMA,
            pltpu.SemaphoreType.DMA,
            pltpu.SemaphoreType.DMA,
        ],
    )
    return kfn(table, edge_index)


def kernel(node_embedding_matrix, batch_x_index, edge_index):
    del batch_x_index
    return _run(node_embedding_matrix, edge_index)
